# Initial kernel scaffold; baseline (speedup 1.0000x reference)
import jax
import jax.numpy as jnp
from jax.experimental import pallas as pl


def kernel(node_feat, net_feat, pin_feat, edge_feat, gc_W, gc_b, t_pool_W, t_pool_b, t_neigh_W, t_self_W, t_self_b, g_pool_W, g_pool_b, g_neigh_W, g_self_W, g_self_b, topo_w_W, topo_w_b, geom_w_W, geom_w_b, net_lin_W, net_lin_b, pin_src, pin_dst, near_src, near_dst):
    raise NotImplementedError("write your pallas kernel here")


# speedup vs baseline: 1.0344x; 1.0344x over previous
import functools

import jax
import jax.numpy as jnp
from jax.experimental import pallas as pl


def _mm3_body(a, b, c, wa, wb, wc, bias, o):
    acc = jnp.dot(a[...], wa[...], preferred_element_type=jnp.float32)
    acc += jnp.dot(b[...], wb[...], preferred_element_type=jnp.float32)
    acc += jnp.dot(c[...], wc[...], preferred_element_type=jnp.float32)
    o[...] = acc + bias[...]


def _mm3(a, b, c, wa, wb, wc, bias, block_rows):
    n, d = a.shape
    do = wa.shape[1]
    grid = (n // block_rows,)
    row_spec = pl.BlockSpec((block_rows, d), lambda i: (i, 0))
    w_spec = pl.BlockSpec((d, do), lambda i: (0, 0))
    return pl.pallas_call(
        _mm3_body,
        grid=grid,
        in_specs=[row_spec, row_spec, row_spec, w_spec, w_spec, w_spec,
                  pl.BlockSpec((1, do), lambda i: (0, 0))],
        out_specs=pl.BlockSpec((block_rows, do), lambda i: (i, 0)),
        out_shape=jax.ShapeDtypeStruct((n, do), jnp.float32),
    )(a, b, c, wa, wb, wc, bias.reshape(1, do))


def _relu_mm_body(x, w, bias, o):
    o[...] = jax.nn.relu(
        jnp.dot(x[...], w[...], preferred_element_type=jnp.float32) + bias[...])


def _relu_mm(x, w, bias, block_rows):
    n, d = x.shape
    do = w.shape[1]
    return pl.pallas_call(
        _relu_mm_body,
        grid=(n // block_rows,),
        in_specs=[pl.BlockSpec((block_rows, d), lambda i: (i, 0)),
                  pl.BlockSpec((d, do), lambda i: (0, 0)),
                  pl.BlockSpec((1, do), lambda i: (0, 0))],
        out_specs=pl.BlockSpec((block_rows, do), lambda i: (i, 0)),
        out_shape=jax.ShapeDtypeStruct((n, do), jnp.float32),
    )(x, w, bias.reshape(1, do))


def kernel(node_feat, net_feat, pin_feat, edge_feat, gc_W, gc_b, t_pool_W, t_pool_b,
           t_neigh_W, t_self_W, t_self_b, g_pool_W, g_pool_b, g_neigh_W, g_self_W,
           g_self_b, topo_w_W, topo_w_b, geom_w_W, geom_w_b, net_lin_W, net_lin_b,
           pin_src, pin_dst, near_src, near_dst):
    N_CELL, _ = node_feat.shape
    N_NET, _ = net_feat.shape
    E_PIN = pin_src.shape[0]
    E_NEAR = near_src.shape[0]

    ew_pin = jax.nn.sigmoid(pin_feat @ topo_w_W + topo_w_b)
    ew_near = jax.nn.sigmoid(edge_feat @ geom_w_W + geom_w_b)

    ones_pin = jnp.ones((E_PIN,), jnp.float32)
    ones_near = jnp.ones((E_NEAR,), jnp.float32)

    deg_src = jax.ops.segment_sum(ones_pin, pin_src, N_CELL)
    norm_src = jnp.where(deg_src > 0, deg_src ** -0.5, 0.0)
    feat = node_feat * norm_src[:, None]
    agg = jax.ops.segment_sum(feat[pin_src], pin_dst, N_NET)
    deg_dst = jax.ops.segment_sum(ones_pin, pin_dst, N_NET)
    norm_dst = jnp.where(deg_dst > 0, deg_dst ** -0.5, 0.0)

    h_pool = _relu_mm(net_feat, t_pool_W, t_pool_b, 1000)
    m = h_pool[pin_dst] * ew_pin
    neigh = jax.ops.segment_max(m, pin_src, N_CELL)
    neigh = jnp.where(deg_src[:, None] > 0, neigh, 0.0)

    h_pool2 = _relu_mm(node_feat, g_pool_W, g_pool_b, 1000)
    m2 = h_pool2[near_src] * ew_near
    neigh2 = jax.ops.segment_max(m2, near_dst, N_CELL)
    has2 = jax.ops.segment_sum(ones_near, near_dst, N_CELL) > 0
    neigh2 = jnp.where(has2[:, None], neigh2, 0.0)

    cell_out = _mm3(node_feat, neigh, neigh2,
                    t_self_W + g_self_W, t_neigh_W, g_neigh_W,
                    t_self_b + g_self_b, 1000)

    aggn = agg * norm_dst[:, None]
    net_out = _mm3(aggn, net_feat, jnp.zeros_like(net_feat),
                   gc_W, net_lin_W, jnp.zeros_like(net_lin_W),
                   gc_b + net_lin_b, 1000)
    return (cell_out, net_out)


# speedup vs baseline: 1.2108x; 1.1705x over previous
import functools

import jax
import jax.numpy as jnp
from jax import lax
from jax.experimental import pallas as pl
from jax.experimental.pallas import tpu as pltpu
from jax.experimental.pallas import tpu_sc as plsc


def _mm3_body(a, b, c, wa, wb, wc, bias, o):
    acc = jnp.dot(a[...], wa[...], preferred_element_type=jnp.float32)
    acc += jnp.dot(b[...], wb[...], preferred_element_type=jnp.float32)
    acc += jnp.dot(c[...], wc[...], preferred_element_type=jnp.float32)
    o[...] = acc + bias[...]


def _mm3(a, b, c, wa, wb, wc, bias, block_rows):
    n, d = a.shape
    do = wa.shape[1]
    row_spec = pl.BlockSpec((block_rows, d), lambda i: (i, 0))
    w_spec = pl.BlockSpec((d, do), lambda i: (0, 0))
    return pl.pallas_call(
        _mm3_body,
        grid=(n // block_rows,),
        in_specs=[row_spec, row_spec, row_spec, w_spec, w_spec, w_spec,
                  pl.BlockSpec((1, do), lambda i: (0, 0))],
        out_specs=pl.BlockSpec((block_rows, do), lambda i: (i, 0)),
        out_shape=jax.ShapeDtypeStruct((n, do), jnp.float32),
    )(a, b, c, wa, wb, wc, bias.reshape(1, do))


def _relu_mm_body(x, w, bias, o):
    o[...] = jax.nn.relu(
        jnp.dot(x[...], w[...], preferred_element_type=jnp.float32) + bias[...])


def _relu_mm(x, w, bias, block_rows):
    n, d = x.shape
    do = w.shape[1]
    return pl.pallas_call(
        _relu_mm_body,
        grid=(n // block_rows,),
        in_specs=[pl.BlockSpec((block_rows, d), lambda i: (i, 0)),
                  pl.BlockSpec((d, do), lambda i: (0, 0)),
                  pl.BlockSpec((1, do), lambda i: (0, 0))],
        out_specs=pl.BlockSpec((block_rows, do), lambda i: (i, 0)),
        out_shape=jax.ShapeDtypeStruct((n, do), jnp.float32),
    )(x, w, bias.reshape(1, do))


_NW = 32
_DAUG = 80


def _make_segsum(n_rows_padded, k_chunks, n_table):
    mesh = plsc.VectorSubcoreMesh(core_axis_name="c", subcore_axis_name="s")
    rows_per_sub = n_rows_padded // 16

    @functools.partial(
        pl.kernel, mesh=mesh,
        compiler_params=pltpu.CompilerParams(use_tc_tiling_on_sc=False),
        out_type=jax.ShapeDtypeStruct((2, n_rows_padded, _DAUG), jnp.float32),
        scratch_types=[
            pltpu.VMEM((k_chunks, 128), jnp.int32),
            pltpu.VMEM((k_chunks, 128), jnp.int32),
            pltpu.VMEM((2, 128, _DAUG), jnp.float32),
            pltpu.VMEM_SHARED((n_rows_padded, _DAUG), jnp.float32),
            pltpu.SemaphoreType.DMA,
            pltpu.SemaphoreType.DMA,
        ],
    )
    def seg_sum(table_hbm, src_hbm, dst_hbm, out_hbm,
                src_v, dst_v, rows_v, acc_sh, gsem, ssem):
        cid = lax.axis_index("c")
        sid = lax.axis_index("s")

        zrow = jnp.zeros((16,), jnp.float32)

        def zero_body(i, _):
            for q in range(_DAUG // 16):
                rows_v[0, i, pl.ds(q * 16, 16)] = zrow
            return 0

        lax.fori_loop(0, 128, zero_body, 0)
        base = sid * rows_per_sub
        nfull = rows_per_sub // 128
        for z in range(nfull):
            pltpu.sync_copy(rows_v.at[0],
                            acc_sh.at[pl.ds(base + z * 128, 128)])
        rem = rows_per_sub - nfull * 128
        if rem:
            pltpu.sync_copy(rows_v.at[0, pl.ds(0, rem)],
                            acc_sh.at[pl.ds(base + nfull * 128, rem)])
        plsc.subcore_barrier()

        pltpu.sync_copy(src_hbm.at[sid], src_v)
        pltpu.sync_copy(dst_hbm.at[sid], dst_v)
        off = (cid * n_table).astype(jnp.int32)

        def bump_body(i, _):
            r = lax.div(i, jnp.int32(8))
            q = lax.rem(i, jnp.int32(8))
            src_v[r, pl.ds(q * 16, 16)] = src_v[r, pl.ds(q * 16, 16)] + off
            return 0

        lax.fori_loop(0, k_chunks * 8, bump_body, 0)

        def chunk_body(j, _):
            slot = lax.rem(j, 2)
            pltpu.async_copy(table_hbm.at[src_v.at[j]], rows_v.at[slot],
                             gsem).wait()
            pltpu.sync_copy(rows_v.at[slot], acc_sh.at[dst_v.at[j]],
                            add=True)
            return 0

        lax.fori_loop(0, k_chunks, chunk_body, 0)
        plsc.subcore_barrier()

        for z in range(nfull):
            pltpu.sync_copy(acc_sh.at[pl.ds(base + z * 128, 128)],
                            rows_v.at[1])
            pltpu.sync_copy(rows_v.at[1],
                            out_hbm.at[cid, pl.ds(base + z * 128, 128)])
        if rem:
            pltpu.sync_copy(acc_sh.at[pl.ds(base + nfull * 128, rem)],
                            rows_v.at[1, pl.ds(0, rem)])
            pltpu.sync_copy(rows_v.at[1, pl.ds(0, rem)],
                            out_hbm.at[cid, pl.ds(base + nfull * 128, rem)])

    return seg_sum


def kernel(node_feat, net_feat, pin_feat, edge_feat, gc_W, gc_b, t_pool_W, t_pool_b,
           t_neigh_W, t_self_W, t_self_b, g_pool_W, g_pool_b, g_neigh_W, g_self_W,
           g_self_b, topo_w_W, topo_w_b, geom_w_W, geom_w_b, net_lin_W, net_lin_b,
           pin_src, pin_dst, near_src, near_dst):
    N_CELL, _ = node_feat.shape
    N_NET, _ = net_feat.shape
    E_PIN = pin_src.shape[0]
    E_NEAR = near_src.shape[0]

    ew_pin = jax.nn.sigmoid(pin_feat @ topo_w_W + topo_w_b)
    ew_near = jax.nn.sigmoid(edge_feat @ geom_w_W + geom_w_b)

    ones_pin = jnp.ones((E_PIN,), jnp.float32)
    ones_near = jnp.ones((E_NEAR,), jnp.float32)

    deg_src = jax.ops.segment_sum(ones_pin, pin_src, N_CELL)
    norm_src = jnp.where(deg_src > 0, deg_src ** -0.5, 0.0)
    feat = node_feat * norm_src[:, None]

    n_net_pad = 10112
    k_chunks = -(-E_PIN // (16 * 128))
    e_pad = 16 * k_chunks * 128
    src_p = jnp.concatenate(
        [pin_src, jnp.zeros((e_pad - E_PIN,), jnp.int32)]).reshape(16, k_chunks, 128)
    dst_p = jnp.concatenate(
        [pin_dst, jnp.full((e_pad - E_PIN,), n_net_pad - 1, jnp.int32)]
    ).reshape(16, k_chunks, 128)
    zc = jnp.zeros((N_CELL, _DAUG - 65), jnp.float32)
    table = jnp.concatenate([
        jnp.concatenate([feat[:, :64], jnp.ones((N_CELL, 1), jnp.float32), zc], 1),
        jnp.concatenate([feat[:, 64:], jnp.zeros((N_CELL, 1), jnp.float32), zc], 1),
    ], axis=0)
    parts = _make_segsum(n_net_pad, k_chunks, N_CELL)(table, src_p, dst_p)
    agg = jnp.concatenate([parts[0, :N_NET, :64], parts[1, :N_NET, :64]], axis=1)
    deg_dst = parts[0, :N_NET, 64]
    norm_dst = jnp.where(deg_dst > 0, deg_dst ** -0.5, 0.0)

    h_pool = _relu_mm(net_feat, t_pool_W, t_pool_b, 1000)
    m = h_pool[pin_dst] * ew_pin
    neigh = jax.ops.segment_max(m, pin_src, N_CELL)
    neigh = jnp.where(deg_src[:, None] > 0, neigh, 0.0)

    h_pool2 = _relu_mm(node_feat, g_pool_W, g_pool_b, 1000)
    m2 = h_pool2[near_src] * ew_near
    neigh2 = jax.ops.segment_max(m2, near_dst, N_CELL)
    has2 = jax.ops.segment_sum(ones_near, near_dst, N_CELL) > 0
    neigh2 = jnp.where(has2[:, None], neigh2, 0.0)

    cell_out = _mm3(node_feat, neigh, neigh2,
                    t_self_W + g_self_W, t_neigh_W, g_neigh_W,
                    t_self_b + g_self_b, 1000)

    aggn = agg * norm_dst[:, None]
    net_out = _mm3(aggn, net_feat, jnp.zeros_like(net_feat),
                   gc_W, net_lin_W, jnp.zeros_like(net_lin_W),
                   gc_b + net_lin_b, 1000)
    return (cell_out, net_out)
